# baseline (speedup 1.0000x reference)
import jax
import jax.numpy as jnp
from jax.experimental import pallas as pl
from jax.experimental.pallas import tpu as pltpu

E = 64
TOP_K = 2
D = 768
F = 1024
FH = F // 2
S = 2048
BLK = 128


def _router_kernel(h_ref, gw_ref, idx_ref, wgt_ref, meta_ref):
    h = h_ref[...]
    gw = gw_ref[...]
    logits = jax.lax.dot_general(h, gw, (((1,), (1,)), ((), ())),
                                 preferred_element_type=jnp.float32)
    p = jax.nn.softmax(logits, axis=-1)

    lane = jax.lax.broadcasted_iota(jnp.int32, (S, E), 1)
    m0 = jnp.max(p, axis=-1, keepdims=True)
    e0 = jnp.min(jnp.where(p == m0, lane, E), axis=-1)
    p_masked = jnp.where(lane == e0[:, None], -1.0, p)
    m1 = jnp.max(p_masked, axis=-1, keepdims=True)
    e1 = jnp.min(jnp.where(p_masked == m1, lane, E), axis=-1)
    p0 = m0[:, 0]
    p1 = m1[:, 0]
    denom = p0 + p1
    w0 = p0 / denom
    w1 = p1 / denom

    oh0 = (lane == e0[:, None]).astype(jnp.float32)
    oh1 = (lane == e1[:, None]).astype(jnp.float32)
    occ = oh0 + oh1
    ti = jax.lax.broadcasted_iota(jnp.int32, (S, S), 0)
    tj = jax.lax.broadcasted_iota(jnp.int32, (S, S), 1)
    tril = (tj <= ti).astype(jnp.float32)
    csum = jax.lax.dot_general(tril, occ, (((1,), (0,)), ((), ())),
                               preferred_element_type=jnp.float32)
    r0 = jnp.sum(csum * oh0, axis=-1) - 1.0
    r1 = jnp.sum(csum * oh1, axis=-1) - 1.0

    counts = csum[S - 1, :]
    nsub = jnp.floor((counts + (BLK - 1)) / BLK)

    zi = jnp.zeros((S,), jnp.int32)
    idx_ref[...] = jnp.stack([e0, e1,
                              r0.astype(jnp.int32), r1.astype(jnp.int32),
                              zi, zi, zi, zi], axis=0)
    wz = jnp.zeros((S,), jnp.float32)
    wgt_ref[...] = jnp.stack([w0, w1, wz, wz, wz, wz, wz, wz], axis=0)
    nsub_p = jnp.concatenate([nsub.astype(jnp.int32), zi[:E]])
    mz = jnp.zeros((128,), jnp.int32)
    meta_ref[...] = jnp.stack([nsub_p, mz, mz, mz, mz, mz, mz, mz], axis=0)


def _moe_kernel(nsub_ref, h_ref, idx_ref, wgt_ref,
                wg_ref, wu_ref, wd_ref, out_ref, x_scr, y_scr):
    e = pl.program_id(0)
    j = pl.program_id(1)
    nsub = nsub_ref[e]

    @pl.when((e == 0) & (j == 0))
    def _init():
        out_ref[...] = jnp.zeros_like(out_ref)

    ids = idx_ref[...]
    wts = wgt_ref[...]
    e0 = ids[0:1, :]
    e1 = ids[1:2, :]
    r0 = ids[2:3, :]
    r1 = ids[3:4, :]
    w0 = wts[0:1, :]
    w1 = wts[1:2, :]
    jrow = jax.lax.broadcasted_iota(jnp.int32, (BLK, S), 0)

    def body(k, carry):
        sr = k * BLK
        m0 = (e0 == e) & ((r0 - sr) == jrow)
        m1 = (e1 == e) & ((r1 - sr) == jrow)

        @pl.when(j == 0)
        def _first_half():
            disp = m0.astype(jnp.float32) + m1.astype(jnp.float32)
            x = jax.lax.dot_general(disp, h_ref[...], (((1,), (0,)), ((), ())),
                                    preferred_element_type=jnp.float32)
            x_scr[pl.ds(sr, BLK), :] = x
            g = jax.lax.dot_general(x, wg_ref[0], (((1,), (0,)), ((), ())),
                                    preferred_element_type=jnp.float32)
            u = jax.lax.dot_general(x, wu_ref[0], (((1,), (0,)), ((), ())),
                                    preferred_element_type=jnp.float32)
            a = g * jax.lax.logistic(g) * u
            y_scr[pl.ds(sr, BLK), :] = jax.lax.dot_general(
                a, wd_ref[0], (((1,), (0,)), ((), ())),
                preferred_element_type=jnp.float32)

        @pl.when(j == 1)
        def _second_half():
            x = x_scr[pl.ds(sr, BLK), :]
            g = jax.lax.dot_general(x, wg_ref[0], (((1,), (0,)), ((), ())),
                                    preferred_element_type=jnp.float32)
            u = jax.lax.dot_general(x, wu_ref[0], (((1,), (0,)), ((), ())),
                                    preferred_element_type=jnp.float32)
            a = g * jax.lax.logistic(g) * u
            y = y_scr[pl.ds(sr, BLK), :] + jax.lax.dot_general(
                a, wd_ref[0], (((1,), (0,)), ((), ())),
                preferred_element_type=jnp.float32)
            wdisp = m0.astype(jnp.float32) * w0 + m1.astype(jnp.float32) * w1
            out_ref[...] += jax.lax.dot_general(wdisp, y, (((0,), (0,)), ((), ())),
                                                preferred_element_type=jnp.float32)

        return carry

    jax.lax.fori_loop(0, nsub, body, 0)


@jax.jit
def kernel(hidden_states, gate_w, w_gate_proj, w_up_proj, w_down_proj):
    b, s, d = hidden_states.shape
    h = hidden_states.reshape(s, d)

    idx, wgt, meta = pl.pallas_call(
        _router_kernel,
        out_shape=(
            jax.ShapeDtypeStruct((8, S), jnp.int32),
            jax.ShapeDtypeStruct((8, S), jnp.float32),
            jax.ShapeDtypeStruct((8, 128), jnp.int32),
        ),
    )(h, gate_w)

    nsub = meta[0, :E]

    grid_spec = pltpu.PrefetchScalarGridSpec(
        num_scalar_prefetch=1,
        grid=(E, 2),
        in_specs=[
            pl.BlockSpec((S, D), lambda e, j, ns: (0, 0)),
            pl.BlockSpec((8, S), lambda e, j, ns: (0, 0)),
            pl.BlockSpec((8, S), lambda e, j, ns: (0, 0)),
            pl.BlockSpec((1, D, FH), lambda e, j, ns: (e, 0, j)),
            pl.BlockSpec((1, D, FH), lambda e, j, ns: (e, 0, j)),
            pl.BlockSpec((1, FH, D), lambda e, j, ns: (e, j, 0)),
        ],
        out_specs=pl.BlockSpec((S, D), lambda e, j, ns: (0, 0)),
        scratch_shapes=[
            pltpu.VMEM((S, D), jnp.float32),
            pltpu.VMEM((S, D), jnp.float32),
        ],
    )
    out = pl.pallas_call(
        _moe_kernel,
        grid_spec=grid_spec,
        out_shape=jax.ShapeDtypeStruct((S, D), jnp.float32),
    )(nsub, h, idx, wgt, w_gate_proj, w_up_proj, w_down_proj)

    return out.reshape(b, s, d)


# speedup vs baseline: 1.4775x; 1.4775x over previous
import jax
import jax.numpy as jnp
from jax.experimental import pallas as pl
from jax.experimental.pallas import tpu as pltpu

E = 64
TOP_K = 2
D = 768
F = 1024
S = 2048
BLK = 128
NBUF = 3


def _router_kernel(h_ref, gw_ref, idx_ref, wgt_ref, meta_ref):
    h = h_ref[...]
    gw = gw_ref[...]
    logits = jax.lax.dot_general(h, gw, (((1,), (1,)), ((), ())),
                                 preferred_element_type=jnp.float32)
    p = jax.nn.softmax(logits, axis=-1)

    lane = jax.lax.broadcasted_iota(jnp.int32, (S, E), 1)
    m0 = jnp.max(p, axis=-1, keepdims=True)
    e0 = jnp.min(jnp.where(p == m0, lane, E), axis=-1)
    p_masked = jnp.where(lane == e0[:, None], -1.0, p)
    m1 = jnp.max(p_masked, axis=-1, keepdims=True)
    e1 = jnp.min(jnp.where(p_masked == m1, lane, E), axis=-1)
    p0 = m0[:, 0]
    p1 = m1[:, 0]
    denom = p0 + p1
    w0 = p0 / denom
    w1 = p1 / denom

    oh0 = (lane == e0[:, None]).astype(jnp.float32)
    oh1 = (lane == e1[:, None]).astype(jnp.float32)
    occ = oh0 + oh1
    ti = jax.lax.broadcasted_iota(jnp.int32, (S, S), 0)
    tj = jax.lax.broadcasted_iota(jnp.int32, (S, S), 1)
    tril = (tj <= ti).astype(jnp.float32)
    csum = jax.lax.dot_general(tril, occ, (((1,), (0,)), ((), ())),
                               preferred_element_type=jnp.float32)
    r0 = jnp.sum(csum * oh0, axis=-1) - 1.0
    r1 = jnp.sum(csum * oh1, axis=-1) - 1.0

    counts = csum[S - 1, :]
    nsub = jnp.floor((counts + (BLK - 1)) / BLK)

    zi = jnp.zeros((S,), jnp.int32)
    idx_ref[...] = jnp.stack([e0, e1,
                              r0.astype(jnp.int32), r1.astype(jnp.int32),
                              zi, zi, zi, zi], axis=0)
    wz = jnp.zeros((S,), jnp.float32)
    wgt_ref[...] = jnp.stack([w0, w1, wz, wz, wz, wz, wz, wz], axis=0)
    nsub_p = jnp.concatenate([nsub.astype(jnp.int32), zi[:E]])
    mz = jnp.zeros((128,), jnp.int32)
    meta_ref[...] = jnp.stack([nsub_p, mz, mz, mz, mz, mz, mz, mz], axis=0)


def _moe_kernel(nsub_ref, h_ref, idx_ref, wgt_ref,
                wg_hbm, wu_hbm, wd_hbm, out_ref,
                wg_buf, wu_buf, wd_buf, sems):
    out_ref[...] = jnp.zeros_like(out_ref)

    ids = idx_ref[...]
    wts = wgt_ref[...]
    e0 = ids[0:1, :]
    e1 = ids[1:2, :]
    r0 = ids[2:3, :]
    r1 = ids[3:4, :]
    w0 = wts[0:1, :]
    w1 = wts[1:2, :]
    jrow = jax.lax.broadcasted_iota(jnp.int32, (BLK, S), 0)
    h = h_ref[...]

    def _copies(e, slot):
        return (
            pltpu.make_async_copy(wg_hbm.at[e], wg_buf.at[slot], sems.at[0, slot]),
            pltpu.make_async_copy(wu_hbm.at[e], wu_buf.at[slot], sems.at[1, slot]),
            pltpu.make_async_copy(wd_hbm.at[e], wd_buf.at[slot], sems.at[2, slot]),
        )

    def _start(e, slot):
        for c in _copies(e, slot):
            c.start()

    def _wait(e, slot):
        for c in _copies(e, slot):
            c.wait()

    for p in range(NBUF):
        _start(p, p)

    def expert_step(e, carry):
        slot = jax.lax.rem(e, NBUF)
        _wait(e, slot)
        nsub = nsub_ref[e]

        def body(k, inner):
            sr = k * BLK
            m0 = (e0 == e) & ((r0 - sr) == jrow)
            m1 = (e1 == e) & ((r1 - sr) == jrow)
            disp = m0.astype(jnp.float32) + m1.astype(jnp.float32)
            x = jax.lax.dot_general(disp, h, (((1,), (0,)), ((), ())),
                                    preferred_element_type=jnp.float32)
            g = jax.lax.dot_general(x, wg_buf[slot], (((1,), (0,)), ((), ())),
                                    preferred_element_type=jnp.float32)
            u = jax.lax.dot_general(x, wu_buf[slot], (((1,), (0,)), ((), ())),
                                    preferred_element_type=jnp.float32)
            a = g * jax.lax.logistic(g) * u
            y = jax.lax.dot_general(a, wd_buf[slot], (((1,), (0,)), ((), ())),
                                    preferred_element_type=jnp.float32)
            wdisp = m0.astype(jnp.float32) * w0 + m1.astype(jnp.float32) * w1
            out_ref[...] += jax.lax.dot_general(
                wdisp, y, (((0,), (0,)), ((), ())),
                preferred_element_type=jnp.float32)
            return inner

        jax.lax.fori_loop(0, nsub, body, 0)

        @pl.when(e + NBUF < E)
        def _refill():
            _start(e + NBUF, slot)

        return carry

    jax.lax.fori_loop(0, E, expert_step, 0)


@jax.jit
def kernel(hidden_states, gate_w, w_gate_proj, w_up_proj, w_down_proj):
    b, s, d = hidden_states.shape
    h = hidden_states.reshape(s, d)

    idx, wgt, meta = pl.pallas_call(
        _router_kernel,
        out_shape=(
            jax.ShapeDtypeStruct((8, S), jnp.int32),
            jax.ShapeDtypeStruct((8, S), jnp.float32),
            jax.ShapeDtypeStruct((8, 128), jnp.int32),
        ),
    )(h, gate_w)

    nsub = meta[0, :E]

    out = pl.pallas_call(
        _moe_kernel,
        in_specs=[
            pl.BlockSpec(memory_space=pltpu.MemorySpace.SMEM),
            pl.BlockSpec(memory_space=pltpu.MemorySpace.VMEM),
            pl.BlockSpec(memory_space=pltpu.MemorySpace.VMEM),
            pl.BlockSpec(memory_space=pltpu.MemorySpace.VMEM),
            pl.BlockSpec(memory_space=pltpu.MemorySpace.HBM),
            pl.BlockSpec(memory_space=pltpu.MemorySpace.HBM),
            pl.BlockSpec(memory_space=pltpu.MemorySpace.HBM),
        ],
        out_specs=pl.BlockSpec(memory_space=pltpu.MemorySpace.VMEM),
        scratch_shapes=[
            pltpu.VMEM((NBUF, D, F), jnp.float32),
            pltpu.VMEM((NBUF, D, F), jnp.float32),
            pltpu.VMEM((NBUF, F, D), jnp.float32),
            pltpu.SemaphoreType.DMA((3, NBUF)),
        ],
        out_shape=jax.ShapeDtypeStruct((S, D), jnp.float32),
    )(nsub, h, idx, wgt, w_gate_proj, w_up_proj, w_down_proj)

    return out.reshape(b, s, d)


# speedup vs baseline: 1.4870x; 1.0064x over previous
import jax
import jax.numpy as jnp
from jax.experimental import pallas as pl
from jax.experimental.pallas import tpu as pltpu

E = 64
TOP_K = 2
D = 768
F = 1024
S = 2048
BLK = 128
NBUF = 4


def _router_kernel(h_ref, gw_ref, idx_ref, wgt_ref, meta_ref):
    h = h_ref[...]
    gw = gw_ref[...]
    logits = jax.lax.dot_general(h, gw, (((1,), (1,)), ((), ())),
                                 preferred_element_type=jnp.float32)
    p = jax.nn.softmax(logits, axis=-1)

    lane = jax.lax.broadcasted_iota(jnp.int32, (S, E), 1)
    m0 = jnp.max(p, axis=-1, keepdims=True)
    e0 = jnp.min(jnp.where(p == m0, lane, E), axis=-1)
    p_masked = jnp.where(lane == e0[:, None], -1.0, p)
    m1 = jnp.max(p_masked, axis=-1, keepdims=True)
    e1 = jnp.min(jnp.where(p_masked == m1, lane, E), axis=-1)
    p0 = m0[:, 0]
    p1 = m1[:, 0]
    denom = p0 + p1
    w0 = p0 / denom
    w1 = p1 / denom

    oh0 = (lane == e0[:, None]).astype(jnp.float32)
    oh1 = (lane == e1[:, None]).astype(jnp.float32)
    occ = oh0 + oh1
    ti = jax.lax.broadcasted_iota(jnp.int32, (S, S), 0)
    tj = jax.lax.broadcasted_iota(jnp.int32, (S, S), 1)
    tril = (tj <= ti).astype(jnp.float32)
    csum = jax.lax.dot_general(tril, occ, (((1,), (0,)), ((), ())),
                               preferred_element_type=jnp.float32)
    r0 = jnp.sum(csum * oh0, axis=-1) - 1.0
    r1 = jnp.sum(csum * oh1, axis=-1) - 1.0

    counts = csum[S - 1, :]
    nsub = jnp.floor((counts + (BLK - 1)) / BLK)

    zi = jnp.zeros((S,), jnp.int32)
    idx_ref[...] = jnp.stack([e0, e1,
                              r0.astype(jnp.int32), r1.astype(jnp.int32),
                              zi, zi, zi, zi], axis=0)
    wz = jnp.zeros((S,), jnp.float32)
    wgt_ref[...] = jnp.stack([w0, w1, wz, wz, wz, wz, wz, wz], axis=0)
    nsub_p = jnp.concatenate([nsub.astype(jnp.int32), zi[:E]])
    mz = jnp.zeros((128,), jnp.int32)
    meta_ref[...] = jnp.stack([nsub_p, mz, mz, mz, mz, mz, mz, mz], axis=0)


def _moe_kernel(nsub_ref, h_ref, idx_ref, wgt_ref,
                wg_hbm, wu_hbm, wd_hbm, out_ref,
                wg_buf, wu_buf, wd_buf, sems):
    out_ref[...] = jnp.zeros_like(out_ref)

    ids = idx_ref[...]
    wts = wgt_ref[...]
    e0 = ids[0:1, :]
    e1 = ids[1:2, :]
    r0 = ids[2:3, :]
    r1 = ids[3:4, :]
    w0 = wts[0:1, :]
    w1 = wts[1:2, :]
    jrow = jax.lax.broadcasted_iota(jnp.int32, (BLK, S), 0)
    h = h_ref[...]

    def _copies(e, slot):
        return (
            pltpu.make_async_copy(wg_hbm.at[e], wg_buf.at[slot], sems.at[0, slot]),
            pltpu.make_async_copy(wu_hbm.at[e], wu_buf.at[slot], sems.at[1, slot]),
            pltpu.make_async_copy(wd_hbm.at[e], wd_buf.at[slot], sems.at[2, slot]),
        )

    def _start(e, slot):
        for c in _copies(e, slot):
            c.start()

    def _wait(e, slot):
        for c in _copies(e, slot):
            c.wait()

    for p in range(NBUF):
        _start(p, p)

    def expert_step(e, carry):
        slot = jax.lax.rem(e, NBUF)
        _wait(e, slot)
        nsub = nsub_ref[e]

        def body(k, inner):
            sr = k * BLK
            m0 = (e0 == e) & ((r0 - sr) == jrow)
            m1 = (e1 == e) & ((r1 - sr) == jrow)
            disp = m0.astype(jnp.float32) + m1.astype(jnp.float32)
            x = jax.lax.dot_general(disp, h, (((1,), (0,)), ((), ())),
                                    preferred_element_type=jnp.float32)
            g = jax.lax.dot_general(x, wg_buf[slot], (((1,), (0,)), ((), ())),
                                    preferred_element_type=jnp.float32)
            u = jax.lax.dot_general(x, wu_buf[slot], (((1,), (0,)), ((), ())),
                                    preferred_element_type=jnp.float32)
            a = g * jax.lax.logistic(g) * u
            y = jax.lax.dot_general(a, wd_buf[slot], (((1,), (0,)), ((), ())),
                                    preferred_element_type=jnp.float32)
            wdisp = m0.astype(jnp.float32) * w0 + m1.astype(jnp.float32) * w1
            out_ref[...] += jax.lax.dot_general(
                wdisp, y, (((0,), (0,)), ((), ())),
                preferred_element_type=jnp.float32)
            return inner

        jax.lax.fori_loop(0, nsub, body, 0)

        @pl.when(e + NBUF < E)
        def _refill():
            _start(e + NBUF, slot)

        return carry

    jax.lax.fori_loop(0, E, expert_step, 0)


@jax.jit
def kernel(hidden_states, gate_w, w_gate_proj, w_up_proj, w_down_proj):
    b, s, d = hidden_states.shape
    h = hidden_states.reshape(s, d)

    idx, wgt, meta = pl.pallas_call(
        _router_kernel,
        out_shape=(
            jax.ShapeDtypeStruct((8, S), jnp.int32),
            jax.ShapeDtypeStruct((8, S), jnp.float32),
            jax.ShapeDtypeStruct((8, 128), jnp.int32),
        ),
    )(h, gate_w)

    nsub = meta[0, :E]

    out = pl.pallas_call(
        _moe_kernel,
        in_specs=[
            pl.BlockSpec(memory_space=pltpu.MemorySpace.SMEM),
            pl.BlockSpec(memory_space=pltpu.MemorySpace.VMEM),
            pl.BlockSpec(memory_space=pltpu.MemorySpace.VMEM),
            pl.BlockSpec(memory_space=pltpu.MemorySpace.VMEM),
            pl.BlockSpec(memory_space=pltpu.MemorySpace.HBM),
            pl.BlockSpec(memory_space=pltpu.MemorySpace.HBM),
            pl.BlockSpec(memory_space=pltpu.MemorySpace.HBM),
        ],
        out_specs=pl.BlockSpec(memory_space=pltpu.MemorySpace.VMEM),
        scratch_shapes=[
            pltpu.VMEM((NBUF, D, F), jnp.float32),
            pltpu.VMEM((NBUF, D, F), jnp.float32),
            pltpu.VMEM((NBUF, F, D), jnp.float32),
            pltpu.SemaphoreType.DMA((3, NBUF)),
        ],
        out_shape=jax.ShapeDtypeStruct((S, D), jnp.float32),
    )(nsub, h, idx, wgt, w_gate_proj, w_up_proj, w_down_proj)

    return out.reshape(b, s, d)


# speedup vs baseline: 1.5837x; 1.0650x over previous
import jax
import jax.numpy as jnp
from jax.experimental import pallas as pl
from jax.experimental.pallas import tpu as pltpu

E = 64
TOP_K = 2
D = 768
F = 1024
S = 2048
BLK = 128
NBUF = 4
CHUNK = 512


def _moe_kernel(h_ref, gw_ref, wg_hbm, wu_hbm, wd_hbm, out_ref,
                wg_buf, wu_buf, wd_buf, sems,
                nsub_vmem, nsub_smem, nsub_sem, route_i, route_w):
    def _copies(e, slot):
        return (
            pltpu.make_async_copy(wg_hbm.at[e], wg_buf.at[slot], sems.at[0, slot]),
            pltpu.make_async_copy(wu_hbm.at[e], wu_buf.at[slot], sems.at[1, slot]),
            pltpu.make_async_copy(wd_hbm.at[e], wd_buf.at[slot], sems.at[2, slot]),
        )

    def _start(e, slot):
        for c in _copies(e, slot):
            c.start()

    def _wait(e, slot):
        for c in _copies(e, slot):
            c.wait()

    for p in range(NBUF):
        _start(p, p)

    out_ref[...] = jnp.zeros_like(out_ref)

    h = h_ref[...]
    gw = gw_ref[...]
    logits = jax.lax.dot_general(h, gw, (((1,), (1,)), ((), ())),
                                 preferred_element_type=jnp.float32)
    p = jax.nn.softmax(logits, axis=-1)

    lane = jax.lax.broadcasted_iota(jnp.int32, (S, E), 1)
    m0 = jnp.max(p, axis=-1, keepdims=True)
    ex0 = jnp.min(jnp.where(p == m0, lane, E), axis=-1)
    p_masked = jnp.where(lane == ex0[:, None], -1.0, p)
    m1 = jnp.max(p_masked, axis=-1, keepdims=True)
    ex1 = jnp.min(jnp.where(p_masked == m1, lane, E), axis=-1)
    p0 = m0[:, 0]
    p1 = m1[:, 0]
    denom = p0 + p1
    w0 = (p0 / denom)[None, :]
    w1 = (p1 / denom)[None, :]

    oh0 = (lane == ex0[:, None]).astype(jnp.float32)
    oh1 = (lane == ex1[:, None]).astype(jnp.float32)
    occ = oh0 + oh1
    ti = jax.lax.broadcasted_iota(jnp.int32, (CHUNK, CHUNK), 0)
    tj = jax.lax.broadcasted_iota(jnp.int32, (CHUNK, CHUNK), 1)
    tril = (tj <= ti).astype(jnp.float32)
    chunks = []
    prefix = jnp.zeros((1, E), jnp.float32)
    for c in range(S // CHUNK):
        blk = occ[c * CHUNK:(c + 1) * CHUNK, :]
        cs = jax.lax.dot_general(tril, blk, (((1,), (0,)), ((), ())),
                                 preferred_element_type=jnp.float32) + prefix
        prefix = cs[CHUNK - 1:CHUNK, :]
        chunks.append(cs)
    csum = jnp.concatenate(chunks, axis=0)
    r0 = (jnp.sum(csum * oh0, axis=-1) - 1.0).astype(jnp.int32)[None, :]
    r1 = (jnp.sum(csum * oh1, axis=-1) - 1.0).astype(jnp.int32)[None, :]
    e0 = ex0[None, :]
    e1 = ex1[None, :]

    counts = csum[S - 1, :]
    nsub = jnp.floor((counts + (BLK - 1)) / BLK)
    nsub_vmem[...] = jnp.broadcast_to(nsub.astype(jnp.int32)[None, :], (8, E))
    nsub_copy = pltpu.make_async_copy(nsub_vmem, nsub_smem, nsub_sem)
    nsub_copy.start()

    zi = jnp.zeros((1, S), jnp.int32)
    route_i[...] = jnp.concatenate([e0, e1, r0, r1, zi, zi, zi, zi], axis=0)
    zf = jnp.zeros((1, S), jnp.float32)
    route_w[...] = jnp.concatenate([w0, w1, zf, zf, zf, zf, zf, zf], axis=0)
    nsub_copy.wait()

    def expert_step(e, carry):
        slot = jax.lax.rem(e, NBUF)
        _wait(e, slot)
        ns = nsub_smem[0, e]

        def body(k, inner):
            sr = k * BLK
            jrow = jax.lax.broadcasted_iota(jnp.int32, (BLK, S), 0)
            be0 = route_i[0:1, :]
            be1 = route_i[1:2, :]
            br0 = route_i[2:3, :]
            br1 = route_i[3:4, :]
            bw0 = route_w[0:1, :]
            bw1 = route_w[1:2, :]
            mm0 = (be0 == e) & ((br0 - sr) == jrow)
            mm1 = (be1 == e) & ((br1 - sr) == jrow)
            disp = mm0.astype(jnp.float32) + mm1.astype(jnp.float32)
            x = jax.lax.dot_general(disp, h_ref[...], (((1,), (0,)), ((), ())),
                                    preferred_element_type=jnp.float32)
            g = jax.lax.dot_general(x, wg_buf[slot], (((1,), (0,)), ((), ())),
                                    preferred_element_type=jnp.float32)
            u = jax.lax.dot_general(x, wu_buf[slot], (((1,), (0,)), ((), ())),
                                    preferred_element_type=jnp.float32)
            a = g * jax.lax.logistic(g) * u
            y = jax.lax.dot_general(a, wd_buf[slot], (((1,), (0,)), ((), ())),
                                    preferred_element_type=jnp.float32)
            wdisp = mm0.astype(jnp.float32) * bw0 + mm1.astype(jnp.float32) * bw1
            out_ref[...] += jax.lax.dot_general(
                wdisp, y, (((0,), (0,)), ((), ())),
                preferred_element_type=jnp.float32)
            return inner

        jax.lax.fori_loop(0, ns, body, 0)

        @pl.when(e + NBUF < E)
        def _refill():
            _start(e + NBUF, slot)

        return carry

    jax.lax.fori_loop(0, E, expert_step, 0)


@jax.jit
def kernel(hidden_states, gate_w, w_gate_proj, w_up_proj, w_down_proj):
    b, s, d = hidden_states.shape
    h = hidden_states.reshape(s, d)

    out = pl.pallas_call(
        _moe_kernel,
        in_specs=[
            pl.BlockSpec(memory_space=pltpu.MemorySpace.VMEM),
            pl.BlockSpec(memory_space=pltpu.MemorySpace.VMEM),
            pl.BlockSpec(memory_space=pltpu.MemorySpace.HBM),
            pl.BlockSpec(memory_space=pltpu.MemorySpace.HBM),
            pl.BlockSpec(memory_space=pltpu.MemorySpace.HBM),
        ],
        out_specs=pl.BlockSpec(memory_space=pltpu.MemorySpace.VMEM),
        scratch_shapes=[
            pltpu.VMEM((NBUF, D, F), jnp.float32),
            pltpu.VMEM((NBUF, D, F), jnp.float32),
            pltpu.VMEM((NBUF, F, D), jnp.float32),
            pltpu.SemaphoreType.DMA((3, NBUF)),
            pltpu.VMEM((8, E), jnp.int32),
            pltpu.SMEM((8, E), jnp.int32),
            pltpu.SemaphoreType.DMA,
            pltpu.VMEM((8, S), jnp.int32),
            pltpu.VMEM((8, S), jnp.float32),
        ],
        out_shape=jax.ShapeDtypeStruct((S, D), jnp.float32),
    )(h, gate_w, w_gate_proj, w_up_proj, w_down_proj)

    return out.reshape(b, s, d)
